# baseline (speedup 1.0000x reference)
import functools

import jax
import jax.numpy as jnp
from jax import lax
from jax.experimental import pallas as pl
from jax.experimental.pallas import tpu as pltpu
from jax.experimental.pallas import tpu_sc as plsc

NUM_FIELDS = 26
VOCAB = 100000
DIM = 32
BATCH = 16384

N = BATCH * NUM_FIELDS
NW = 32
CH = 512
CPF = BATCH // CH
NCH = NUM_FIELDS * CPF
CPW = NCH // NW
SUB = 128
NSUB = CH // SUB


def _body(idxt, fixed, tab3, out,
          sumbuf, idxbuf, rows, fixv, nzv, nzall, shared, sem):
    nc = plsc.get_sparse_core_info().num_cores
    sid = lax.axis_index("s")
    wid = sid * nc + lax.axis_index("c")

    pltpu.sync_copy(fixed, fixv)
    fix_lo = fixv[pl.ds(0, 16)]
    fix_hi = fixv[pl.ds(16, 16)]

    def field_nz(f):
        pltpu.sync_copy(
            idxt.at[pl.ds(pl.multiple_of(f * BATCH, BATCH), BATCH)], sumbuf)

        def sbody(i, acc):
            return acc + sumbuf[pl.ds(i * 16, 16)]

        acc = lax.fori_loop(0, BATCH // 16, sbody,
                            jnp.zeros((16,), jnp.int32))
        t = acc[0]
        for i in range(1, 16):
            t = t | acc[i]
        return t

    def publish(f, slot):
        nzbit = field_nz(f)
        nzv[pl.ds(0, 16)] = jnp.full((16,), 1, jnp.int32) * nzbit
        pltpu.sync_copy(nzv, shared.at[pl.ds(slot * 16, 16)])

    publish(sid, sid)

    @pl.when(sid < NUM_FIELDS - 16)
    def _():
        publish(sid + 16, sid + 16)

    plsc.subcore_barrier()
    pltpu.sync_copy(shared, nzall)

    def unit(j, carry):
        u = wid * CPW + j
        f = u // CPF
        b0 = (u % CPF) * CH
        pltpu.sync_copy(
            idxt.at[pl.ds(pl.multiple_of(f * BATCH + b0, CH), CH)], idxbuf)
        tf = tab3.at[f]
        copies = [
            pltpu.async_copy(tf.at[idxbuf.at[pl.ds(p * SUB, SUB)]],
                             rows.at[pl.ds(p * SUB, SUB)], sem)
            for p in range(NSUB)
        ]
        for cp in copies:
            cp.wait()

        v = nzall[pl.ds(0, 16)]
        vf = nzall[pl.ds(f * 16, 16)]

        @pl.when(vf[0] == 0)
        def _():
            def fix(r, cy):
                rows[r, pl.ds(0, 16)] = fix_lo
                rows[r, pl.ds(16, 16)] = fix_hi
                return cy
            lax.fori_loop(0, CH, fix, 0)

        pltpu.sync_copy(rows, out.at[pl.ds(b0, CH), f])
        return carry

    lax.fori_loop(0, CPW, unit, 0)


@jax.jit
def _run(idxt, fixed, tab3):
    mesh = plsc.VectorSubcoreMesh(core_axis_name="c", subcore_axis_name="s")
    kern = functools.partial(
        pl.kernel,
        out_type=jax.ShapeDtypeStruct((BATCH, NUM_FIELDS, DIM), jnp.float32),
        mesh=mesh,
        scratch_types=[
            pltpu.VMEM((BATCH,), jnp.int32),
            pltpu.VMEM((CH,), jnp.int32),
            pltpu.VMEM((CH, DIM), jnp.float32),
            pltpu.VMEM((DIM,), jnp.float32),
            pltpu.VMEM((16,), jnp.int32),
            pltpu.VMEM((32 * 16,), jnp.int32),
            pltpu.VMEM_SHARED((32 * 16,), jnp.int32),
            pltpu.SemaphoreType.DMA,
        ],
        compiler_params=pltpu.CompilerParams(use_tc_tiling_on_sc=False),
    )(_body)
    return kern(idxt, fixed, tab3)


def kernel(sparse_inputs, tables, fixed_vector):
    si = sparse_inputs.astype(jnp.int32)
    idxt = si.T.reshape(N)
    return _run(idxt, fixed_vector, tables)


# speedup vs baseline: 1.0008x; 1.0008x over previous
import functools

import jax
import jax.numpy as jnp
from jax import lax
from jax.experimental import pallas as pl
from jax.experimental.pallas import tpu as pltpu
from jax.experimental.pallas import tpu_sc as plsc

NUM_FIELDS = 26
VOCAB = 100000
DIM = 32
BATCH = 16384

N = BATCH * NUM_FIELDS
NW = 32
CH = 512
CPF = BATCH // CH
NCH = NUM_FIELDS * CPF
CPW = NCH // NW
SUB = 128
NSUB = CH // SUB


def _body(idxt, fixed, tab3, out,
          sumbuf, idxbuf, rows, fixv, nzv, nzall, shared, sem):
    nc = plsc.get_sparse_core_info().num_cores
    sid = lax.axis_index("s")
    wid = sid * nc + lax.axis_index("c")

    pltpu.sync_copy(fixed, fixv)
    fix_lo = fixv[pl.ds(0, 16)]
    fix_hi = fixv[pl.ds(16, 16)]

    def field_nz(f):
        pltpu.sync_copy(
            idxt.at[pl.ds(pl.multiple_of(f * BATCH, BATCH), BATCH)], sumbuf)

        def sbody(i, acc):
            return acc + sumbuf[pl.ds(i * 16, 16)]

        acc = lax.fori_loop(0, BATCH // 16, sbody,
                            jnp.zeros((16,), jnp.int32))
        t = acc[0]
        for i in range(1, 16):
            t = t | acc[i]
        return t

    def publish(f, slot):
        nzbit = field_nz(f)
        nzv[pl.ds(0, 16)] = jnp.full((16,), 1, jnp.int32) * nzbit
        pltpu.sync_copy(nzv, shared.at[pl.ds(slot * 16, 16)])

    publish(sid, sid)

    @pl.when(sid < NUM_FIELDS - 16)
    def _():
        publish(sid + 16, sid + 16)

    plsc.subcore_barrier()
    pltpu.sync_copy(shared, nzall)

    def unit(j, carry):
        u = wid * CPW + j
        f = u // CPF
        b0 = (u % CPF) * CH
        pltpu.sync_copy(
            idxt.at[pl.ds(pl.multiple_of(f * BATCH + b0, CH), CH)], idxbuf)
        tf = tab3.at[f]
        copies = [
            pltpu.async_copy(tf.at[idxbuf.at[pl.ds(p * SUB, SUB)]],
                             rows.at[pl.ds(p * SUB, SUB)], sem)
            for p in range(NSUB)
        ]
        for cp in copies:
            cp.wait()

        vf = nzall[pl.ds(f * 16, 16)]

        @pl.when(vf[0] == 0)
        def _():
            def fix(r, cy):
                rows[r, pl.ds(0, 16)] = fix_lo
                rows[r, pl.ds(16, 16)] = fix_hi
                return cy
            lax.fori_loop(0, CH, fix, 0)

        pltpu.sync_copy(rows, out.at[pl.ds(b0, CH), f])
        return carry

    lax.fori_loop(0, CPW, unit, 0)


@jax.jit
def _run(idxt, fixed, tab3):
    mesh = plsc.VectorSubcoreMesh(core_axis_name="c", subcore_axis_name="s")
    kern = functools.partial(
        pl.kernel,
        out_type=jax.ShapeDtypeStruct((BATCH, NUM_FIELDS, DIM), jnp.float32),
        mesh=mesh,
        scratch_types=[
            pltpu.VMEM((BATCH,), jnp.int32),
            pltpu.VMEM((CH,), jnp.int32),
            pltpu.VMEM((CH, DIM), jnp.float32),
            pltpu.VMEM((DIM,), jnp.float32),
            pltpu.VMEM((16,), jnp.int32),
            pltpu.VMEM((32 * 16,), jnp.int32),
            pltpu.VMEM_SHARED((32 * 16,), jnp.int32),
            pltpu.SemaphoreType.DMA,
        ],
        compiler_params=pltpu.CompilerParams(use_tc_tiling_on_sc=False),
    )(_body)
    return kern(idxt, fixed, tab3)


def kernel(sparse_inputs, tables, fixed_vector):
    si = sparse_inputs.astype(jnp.int32)
    idxt = si.T.reshape(N)
    return _run(idxt, fixed_vector, tables)


# speedup vs baseline: 1.0082x; 1.0073x over previous
import functools

import jax
import jax.numpy as jnp
from jax import lax
from jax.experimental import pallas as pl
from jax.experimental.pallas import tpu as pltpu
from jax.experimental.pallas import tpu_sc as plsc

NUM_FIELDS = 26
VOCAB = 100000
DIM = 32
BATCH = 16384

N = BATCH * NUM_FIELDS
NW = 32
BPW = BATCH // NW
BCH = 32
CPW = BPW // BCH
CH = BCH * NUM_FIELDS
SUB = 128
NFULL = CH // SUB
TAIL = CH - NFULL * SUB
SUMW = 128
FPS = BATCH // SUMW


def _body(idxrm, idxt, fixed, tab2, out,
          sumbuf, idxbuf, gidxbuf, rows, fixv, nzv, nzall, shared, sem, sem2):
    nc = plsc.get_sparse_core_info().num_cores
    sid = lax.axis_index("s")
    wid = sid * nc + lax.axis_index("c")

    pltpu.sync_copy(fixed, fixv)
    fix_lo = fixv[pl.ds(0, 16)]
    fix_hi = fixv[pl.ds(16, 16)]

    def field_nz(f):
        pltpu.sync_copy(
            idxt.at[pl.ds(pl.multiple_of(f * FPS, FPS), FPS)], sumbuf)

        def sbody(i, acc):
            return acc + sumbuf[i // 8, pl.ds((i % 8) * 16, 16)]

        acc = lax.fori_loop(0, BATCH // 16, sbody,
                            jnp.zeros((16,), jnp.int32))
        t = acc[0]
        for i in range(1, 16):
            t = t | acc[i]
        return t

    def publish(f, slot):
        nzbit = field_nz(f)
        nzv[pl.ds(0, 16)] = jnp.full((16,), 1, jnp.int32) * nzbit
        pltpu.sync_copy(nzv, shared.at[pl.ds(slot * 16, 16)])

    publish(sid, sid)

    @pl.when(sid < NUM_FIELDS - 16)
    def _():
        publish(sid + 16, sid + 16)

    plsc.subcore_barrier()
    pltpu.sync_copy(shared, nzall)

    m = nzall[pl.ds(0, 16)]
    for f in range(1, NUM_FIELDS):
        m = jnp.minimum(m, jnp.abs(nzall[pl.ds(f * 16, 16)]))
    any_masked = m[0] == 0

    def unit(j, carry):
        bbase = wid * BPW + j * BCH
        base = pl.multiple_of(bbase * NUM_FIELDS, 16)
        pltpu.sync_copy(idxrm.at[pl.ds(base, CH)], idxbuf)
        for g in range(CH // 16):
            pos = lax.iota(jnp.int32, 16) + (base + g * 16)
            offs = lax.rem(pos, NUM_FIELDS) * VOCAB
            gidxbuf[pl.ds(g * 16, 16)] = idxbuf[pl.ds(g * 16, 16)] + offs
        copies = [
            pltpu.async_copy(tab2.at[gidxbuf.at[pl.ds(p * SUB, SUB)]],
                             rows.at[pl.ds(p * SUB, SUB)], sem)
            for p in range(NFULL)
        ]
        copies.append(
            pltpu.async_copy(tab2.at[gidxbuf.at[pl.ds(NFULL * SUB, TAIL)]],
                             rows.at[pl.ds(NFULL * SUB, TAIL)], sem))
        for cp in copies:
            cp.wait()

        @pl.when(any_masked)
        def _():
            def fix(r, cy):
                fr = lax.rem(base + r, NUM_FIELDS)
                v = nzall[pl.ds(fr * 16, 16)]
                msk = v[0] == 0
                lo = rows[r, pl.ds(0, 16)]
                hi = rows[r, pl.ds(16, 16)]
                rows[r, pl.ds(0, 16)] = jnp.where(msk, fix_lo, lo)
                rows[r, pl.ds(16, 16)] = jnp.where(msk, fix_hi, hi)
                return cy
            lax.fori_loop(0, CH, fix, 0)

        outcps = [
            pltpu.async_copy(rows.at[pl.ds(bl * NUM_FIELDS, NUM_FIELDS)],
                             out.at[bbase + bl], sem2)
            for bl in range(BCH)
        ]
        for cp in outcps:
            cp.wait()
        return carry

    lax.fori_loop(0, CPW, unit, 0)


@jax.jit
def _run(idxrm, idxt, fixed, tab2):
    mesh = plsc.VectorSubcoreMesh(core_axis_name="c", subcore_axis_name="s")
    kern = functools.partial(
        pl.kernel,
        out_type=jax.ShapeDtypeStruct((BATCH, NUM_FIELDS, DIM), jnp.float32),
        mesh=mesh,
        scratch_types=[
            pltpu.VMEM((FPS, SUMW), jnp.int32),
            pltpu.VMEM((CH,), jnp.int32),
            pltpu.VMEM((CH,), jnp.int32),
            pltpu.VMEM((CH, DIM), jnp.float32),
            pltpu.VMEM((DIM,), jnp.float32),
            pltpu.VMEM((16,), jnp.int32),
            pltpu.VMEM((32 * 16,), jnp.int32),
            pltpu.VMEM_SHARED((32 * 16,), jnp.int32),
            pltpu.SemaphoreType.DMA,
            pltpu.SemaphoreType.DMA,
        ],
        compiler_params=pltpu.CompilerParams(use_tc_tiling_on_sc=False),
    )(_body)
    return kern(idxrm, idxt, fixed, tab2)


def kernel(sparse_inputs, tables, fixed_vector):
    si = sparse_inputs.astype(jnp.int32)
    idxrm = si.reshape(N)
    idxt = si.T.reshape(NUM_FIELDS * FPS, SUMW)
    tab2 = tables.reshape(NUM_FIELDS * VOCAB, DIM)
    return _run(idxrm, idxt, fixed_vector, tab2)
